# baseline (speedup 1.0000x reference)
import functools

import jax
import jax.numpy as jnp
from jax import lax
from jax.experimental import pallas as pl
from jax.experimental.pallas import tpu as pltpu
from jax.experimental.pallas import tpu_sc as plsc

_NC = 2
_NS = 16
_L = 16

_BAND = 128
_NRANGE = 8
_CCOLS = 368

_TC_BLK = 4096
_SC_COLS = 10 * _TC_BLK


def _argmax_sc(probs_t, vocab, num_rows):
    nbands = num_rows // _BAND
    range_cols = -(-vocab // _NRANGE)
    range_cols = -(-range_cols // 8) * 8
    nchunk = -(-range_cols // _CCOLS)
    nchunk += nchunk % 2
    last_start = vocab - _CCOLS
    npart = _NRANGE * num_rows
    mesh = plsc.VectorSubcoreMesh(
        core_axis_name="c", subcore_axis_name="s",
        num_cores=_NC, num_subcores=_NS)

    @functools.partial(
        pl.kernel,
        out_type=(jax.ShapeDtypeStruct((npart,), jnp.float32),
                  jax.ShapeDtypeStruct((npart,), jnp.int32)),
        mesh=mesh,
        scratch_types=[
            pltpu.VMEM((_CCOLS, _BAND), jnp.float32),
            pltpu.VMEM((_CCOLS, _BAND), jnp.float32),
            pltpu.VMEM((_BAND,), jnp.float32),
            pltpu.VMEM((_BAND,), jnp.int32),
            pltpu.SemaphoreType.DMA,
            pltpu.SemaphoreType.DMA,
        ],
        compiler_params=pltpu.CompilerParams(use_tc_tiling_on_sc=True),
    )
    def body(pt_hbm, out_val, out_idx, buf0, buf1, stgv, stgi, sem0, sem1):
        c = lax.axis_index("c")
        s = lax.axis_index("s")
        band = nbands // _NC * c + s // _NRANGE
        rng = s % _NRANGE
        col0 = rng * range_cols
        lane = lax.broadcasted_iota(jnp.int32, (_L,), 0)
        neg = jnp.full((_L,), -1.0, jnp.float32)
        zero = jnp.zeros((_L,), jnp.int32)
        ngroups = _BAND // _L

        def chunk_src(k):
            start = jnp.minimum(col0 + k * _CCOLS, last_start)
            return pt_hbm.at[pl.ds(start, _CCOLS), pl.ds(band * _BAND, _BAND)]

        pltpu.async_copy(chunk_src(0), buf0, sem0)

        def scan_chunk(k, buf, sem, nbuf, nsem, accs):
            pltpu.async_copy(chunk_src(k + 1), nbuf, nsem)
            pltpu.make_async_copy(chunk_src(k), buf, sem).wait()
            cbase = jnp.minimum(col0 + k * _CCOLS, last_start)

            def col_body(cc, a2):
                vs = list(a2[:ngroups])
                vi = list(a2[ngroups:])
                colidx = cbase + cc
                for r in range(ngroups):
                    x = buf[cc, pl.ds(r * _L, _L)]
                    m = x > vs[r]
                    vs[r] = jnp.where(m, x, vs[r])
                    vi[r] = jnp.where(
                        m, jnp.broadcast_to(colidx, (_L,)), vi[r])
                return tuple(vs) + tuple(vi)

            return lax.fori_loop(0, _CCOLS, col_body, accs)

        def pair_body(kk, accs):
            accs = scan_chunk(2 * kk, buf0, sem0, buf1, sem1, accs)
            accs = scan_chunk(2 * kk + 1, buf1, sem1, buf0, sem0, accs)
            return accs

        init = tuple([neg] * ngroups) + tuple([zero] * ngroups)
        accs = lax.fori_loop(0, nchunk // 2, pair_body, init)
        pltpu.make_async_copy(chunk_src(nchunk), buf0, sem0).wait()
        for r in range(ngroups):
            stgv[pl.ds(r * _L, _L)] = accs[r]
            stgi[pl.ds(r * _L, _L)] = accs[ngroups + r]
        off = rng * num_rows + band * _BAND
        pltpu.sync_copy(stgv, out_val.at[pl.ds(off, _BAND)])
        pltpu.sync_copy(stgi, out_idx.at[pl.ds(off, _BAND)])

    return body(probs_t)


def _argmax_tc(probs_t, col_start, vocab, num_rows):
    grid = -(-(vocab - col_start) // _TC_BLK)

    def body(x_ref, val_ref, idx_ref, vacc, iacc):
        k = pl.program_id(0)
        cols = lax.broadcasted_iota(jnp.int32, (_TC_BLK, num_rows), 0) + (
            col_start + k * _TC_BLK)
        x = jnp.where(cols < vocab, x_ref[...], -1.0)
        bm = jnp.max(x, axis=0, keepdims=True)
        bi = jnp.argmax(x, axis=0, keepdims=True).astype(jnp.int32) + (
            col_start + k * _TC_BLK)

        @pl.when(k == 0)
        def _():
            vacc[...] = bm
            iacc[...] = bi

        @pl.when(k > 0)
        def _():
            better = bm > vacc[...]
            vacc[...] = jnp.where(better, bm, vacc[...])
            iacc[...] = jnp.where(better, bi, iacc[...])

        @pl.when(k == grid - 1)
        def _():
            val_ref[...] = vacc[...]
            idx_ref[...] = iacc[...]

    return pl.pallas_call(
        body,
        grid=(grid,),
        in_specs=[pl.BlockSpec((_TC_BLK, num_rows),
                               lambda k: (col_start // _TC_BLK + k, 0))],
        out_specs=(pl.BlockSpec((1, num_rows), lambda k: (0, 0)),
                   pl.BlockSpec((1, num_rows), lambda k: (0, 0))),
        out_shape=(jax.ShapeDtypeStruct((1, num_rows), jnp.float32),
                   jax.ShapeDtypeStruct((1, num_rows), jnp.int32)),
        scratch_shapes=[pltpu.VMEM((1, num_rows), jnp.float32),
                        pltpu.VMEM((1, num_rows), jnp.int32)],
    )(probs_t)


def _merge_epilogue_tc(val_p, idx_p, val1, idx1, draft, cu2d, bonus,
                       num_rows, spec):
    batch = num_rows // spec
    sentinel = 2 * spec

    def body(v8_ref, i8_ref, v1_ref, i1_ref, tam_ref):
        vs = [v8_ref[pl.ds(r * num_rows, num_rows)] for r in range(_NRANGE)]
        is_ = [i8_ref[pl.ds(r * num_rows, num_rows)] for r in range(_NRANGE)]
        vs.append(v1_ref[...])
        is_.append(i1_ref[...])
        best = vs[0]
        for v in vs[1:]:
            best = jnp.maximum(best, v)
        cand = jnp.where(vs[0] == best, is_[0], 2**30)
        for v, i in zip(vs[1:], is_[1:]):
            cand = jnp.minimum(cand, jnp.where(v == best, i, 2**30))
        tam_ref[...] = cand

    tam = pl.pallas_call(
        body,
        out_shape=jax.ShapeDtypeStruct((num_rows,), jnp.int32),
    )(val_p, idx_p, val1, idx1)

    def body2(tam_ref, draft_ref, cu_ref, bonus_ref, out_ref):
        tam = tam_ref[...]
        draft = draft_ref[...]
        cu = cu_ref[...]
        bns = bonus_ref[...]
        cu_prev = jnp.concatenate(
            [jnp.zeros((1, 1), jnp.int32), cu[:-1, :]], axis=0)
        dpr = cu - cu_prev
        pos = lax.broadcasted_iota(jnp.int32, (batch, spec), 1)
        mm_pos = jnp.where(draft != tam, pos, sentinel)
        fm = jnp.min(mm_pos, axis=1, keepdims=True)
        fm = jnp.where(fm == sentinel, dpr, fm)
        copy_len = jnp.minimum(fm + 1, dpr)
        main = jnp.where(pos < copy_len, tam, -1)
        bcol = jnp.where(fm >= dpr, bns, -1)
        out_ref[...] = jnp.concatenate([main, bcol], axis=1)

    return pl.pallas_call(
        body2,
        out_shape=jax.ShapeDtypeStruct((batch, spec + 1), jnp.int32),
    )(tam.reshape(batch, spec), draft.reshape(batch, spec), cu2d, bonus)


def kernel(draft_token_ids, cu_num_draft_tokens, target_probs,
           bonus_token_ids, max_spec_len):
    num_tokens, vocab = target_probs.shape
    batch = cu_num_draft_tokens.shape[0]
    spec = num_tokens // batch
    probs_t = target_probs.T
    val_p, idx_p = _argmax_sc(probs_t, _SC_COLS, num_tokens)
    val_t, idx_t = _argmax_tc(probs_t, _SC_COLS, vocab, num_tokens)
    return _merge_epilogue_tc(
        val_p, idx_p,
        val_t.reshape(num_tokens), idx_t.reshape(num_tokens),
        draft_token_ids,
        cu_num_draft_tokens.reshape(batch, 1).astype(jnp.int32),
        bonus_token_ids.astype(jnp.int32),
        num_tokens, spec)


# speedup vs baseline: 1.0198x; 1.0198x over previous
import functools

import jax
import jax.numpy as jnp
from jax import lax
from jax.experimental import pallas as pl
from jax.experimental.pallas import tpu as pltpu
from jax.experimental.pallas import tpu_sc as plsc

_NC = 2
_NS = 16
_L = 16

_BAND = 128
_NRANGE = 8
_CCOLS = 368

_TC_BLK = 4096
_SC_COLS = 11 * _TC_BLK


def _argmax_sc(probs_t, vocab, num_rows):
    nbands = num_rows // _BAND
    range_cols = -(-vocab // _NRANGE)
    range_cols = -(-range_cols // 8) * 8
    nchunk = -(-range_cols // _CCOLS)
    nchunk += nchunk % 2
    last_start = vocab - _CCOLS
    npart = _NRANGE * num_rows
    mesh = plsc.VectorSubcoreMesh(
        core_axis_name="c", subcore_axis_name="s",
        num_cores=_NC, num_subcores=_NS)

    @functools.partial(
        pl.kernel,
        out_type=(jax.ShapeDtypeStruct((npart,), jnp.float32),
                  jax.ShapeDtypeStruct((npart,), jnp.int32)),
        mesh=mesh,
        scratch_types=[
            pltpu.VMEM((_CCOLS, _BAND), jnp.float32),
            pltpu.VMEM((_CCOLS, _BAND), jnp.float32),
            pltpu.VMEM((_BAND,), jnp.float32),
            pltpu.VMEM((_BAND,), jnp.int32),
            pltpu.SemaphoreType.DMA,
            pltpu.SemaphoreType.DMA,
        ],
        compiler_params=pltpu.CompilerParams(use_tc_tiling_on_sc=True),
    )
    def body(pt_hbm, out_val, out_idx, buf0, buf1, stgv, stgi, sem0, sem1):
        c = lax.axis_index("c")
        s = lax.axis_index("s")
        band = nbands // _NC * c + s // _NRANGE
        rng = s % _NRANGE
        col0 = rng * range_cols
        lane = lax.broadcasted_iota(jnp.int32, (_L,), 0)
        neg = jnp.full((_L,), -1.0, jnp.float32)
        zero = jnp.zeros((_L,), jnp.int32)
        ngroups = _BAND // _L

        def chunk_src(k):
            start = jnp.minimum(col0 + k * _CCOLS, last_start)
            return pt_hbm.at[pl.ds(start, _CCOLS), pl.ds(band * _BAND, _BAND)]

        pltpu.async_copy(chunk_src(0), buf0, sem0)

        def scan_chunk(k, buf, sem, nbuf, nsem, accs):
            pltpu.async_copy(chunk_src(k + 1), nbuf, nsem)
            pltpu.make_async_copy(chunk_src(k), buf, sem).wait()
            cbase = jnp.minimum(col0 + k * _CCOLS, last_start)

            def col_body(cc, a2):
                vs = list(a2[:ngroups])
                vi = list(a2[ngroups:])
                colidx = cbase + cc
                for r in range(ngroups):
                    x = buf[cc, pl.ds(r * _L, _L)]
                    m = x > vs[r]
                    vs[r] = jnp.where(m, x, vs[r])
                    vi[r] = jnp.where(
                        m, jnp.broadcast_to(colidx, (_L,)), vi[r])
                return tuple(vs) + tuple(vi)

            return lax.fori_loop(0, _CCOLS, col_body, accs)

        def pair_body(kk, accs):
            accs = scan_chunk(2 * kk, buf0, sem0, buf1, sem1, accs)
            accs = scan_chunk(2 * kk + 1, buf1, sem1, buf0, sem0, accs)
            return accs

        init = tuple([neg] * ngroups) + tuple([zero] * ngroups)
        accs = lax.fori_loop(0, nchunk // 2, pair_body, init)
        pltpu.make_async_copy(chunk_src(nchunk), buf0, sem0).wait()
        for r in range(ngroups):
            stgv[pl.ds(r * _L, _L)] = accs[r]
            stgi[pl.ds(r * _L, _L)] = accs[ngroups + r]
        off = rng * num_rows + band * _BAND
        pltpu.sync_copy(stgv, out_val.at[pl.ds(off, _BAND)])
        pltpu.sync_copy(stgi, out_idx.at[pl.ds(off, _BAND)])

    return body(probs_t)


def _argmax_tc(probs_t, col_start, vocab, num_rows):
    grid = -(-(vocab - col_start) // _TC_BLK)

    def body(x_ref, val_ref, idx_ref, vacc, iacc):
        k = pl.program_id(0)
        cols = lax.broadcasted_iota(jnp.int32, (_TC_BLK, num_rows), 0) + (
            col_start + k * _TC_BLK)
        x = jnp.where(cols < vocab, x_ref[...], -1.0)
        bm = jnp.max(x, axis=0, keepdims=True)
        bi = jnp.argmax(x, axis=0, keepdims=True).astype(jnp.int32) + (
            col_start + k * _TC_BLK)

        @pl.when(k == 0)
        def _():
            vacc[...] = bm
            iacc[...] = bi

        @pl.when(k > 0)
        def _():
            better = bm > vacc[...]
            vacc[...] = jnp.where(better, bm, vacc[...])
            iacc[...] = jnp.where(better, bi, iacc[...])

        @pl.when(k == grid - 1)
        def _():
            val_ref[...] = vacc[...]
            idx_ref[...] = iacc[...]

    return pl.pallas_call(
        body,
        grid=(grid,),
        in_specs=[pl.BlockSpec((_TC_BLK, num_rows),
                               lambda k: (col_start // _TC_BLK + k, 0))],
        out_specs=(pl.BlockSpec((1, num_rows), lambda k: (0, 0)),
                   pl.BlockSpec((1, num_rows), lambda k: (0, 0))),
        out_shape=(jax.ShapeDtypeStruct((1, num_rows), jnp.float32),
                   jax.ShapeDtypeStruct((1, num_rows), jnp.int32)),
        scratch_shapes=[pltpu.VMEM((1, num_rows), jnp.float32),
                        pltpu.VMEM((1, num_rows), jnp.int32)],
    )(probs_t)


def _merge_epilogue_tc(val_p, idx_p, val1, idx1, draft, cu2d, bonus,
                       num_rows, spec):
    batch = num_rows // spec
    sentinel = 2 * spec

    def body(v8_ref, i8_ref, v1_ref, i1_ref, tam_ref):
        vs = [v8_ref[pl.ds(r * num_rows, num_rows)] for r in range(_NRANGE)]
        is_ = [i8_ref[pl.ds(r * num_rows, num_rows)] for r in range(_NRANGE)]
        vs.append(v1_ref[...])
        is_.append(i1_ref[...])
        best = vs[0]
        for v in vs[1:]:
            best = jnp.maximum(best, v)
        cand = jnp.where(vs[0] == best, is_[0], 2**30)
        for v, i in zip(vs[1:], is_[1:]):
            cand = jnp.minimum(cand, jnp.where(v == best, i, 2**30))
        tam_ref[...] = cand

    tam = pl.pallas_call(
        body,
        out_shape=jax.ShapeDtypeStruct((num_rows,), jnp.int32),
    )(val_p, idx_p, val1, idx1)

    def body2(tam_ref, draft_ref, cu_ref, bonus_ref, out_ref):
        tam = tam_ref[...]
        draft = draft_ref[...]
        cu = cu_ref[...]
        bns = bonus_ref[...]
        cu_prev = jnp.concatenate(
            [jnp.zeros((1, 1), jnp.int32), cu[:-1, :]], axis=0)
        dpr = cu - cu_prev
        pos = lax.broadcasted_iota(jnp.int32, (batch, spec), 1)
        mm_pos = jnp.where(draft != tam, pos, sentinel)
        fm = jnp.min(mm_pos, axis=1, keepdims=True)
        fm = jnp.where(fm == sentinel, dpr, fm)
        copy_len = jnp.minimum(fm + 1, dpr)
        main = jnp.where(pos < copy_len, tam, -1)
        bcol = jnp.where(fm >= dpr, bns, -1)
        out_ref[...] = jnp.concatenate([main, bcol], axis=1)

    return pl.pallas_call(
        body2,
        out_shape=jax.ShapeDtypeStruct((batch, spec + 1), jnp.int32),
    )(tam.reshape(batch, spec), draft.reshape(batch, spec), cu2d, bonus)


def kernel(draft_token_ids, cu_num_draft_tokens, target_probs,
           bonus_token_ids, max_spec_len):
    num_tokens, vocab = target_probs.shape
    batch = cu_num_draft_tokens.shape[0]
    spec = num_tokens // batch
    probs_t = target_probs.T
    val_p, idx_p = _argmax_sc(probs_t, _SC_COLS, num_tokens)
    val_t, idx_t = _argmax_tc(probs_t, _SC_COLS, vocab, num_tokens)
    return _merge_epilogue_tc(
        val_p, idx_p,
        val_t.reshape(num_tokens), idx_t.reshape(num_tokens),
        draft_token_ids,
        cu_num_draft_tokens.reshape(batch, 1).astype(jnp.int32),
        bonus_token_ids.astype(jnp.int32),
        num_tokens, spec)
